# baseline (speedup 1.0000x reference)
import functools

import jax
import jax.numpy as jnp
from jax import lax
from jax.experimental import pallas as pl
from jax.experimental.pallas import tpu as pltpu
from jax.experimental.pallas import tpu_sc as plsc

N_NUC = 1000
N_ELEC = 9000
N_NODES = 10000
E = 320000
EMB = 128
KER = 64
DFD = 128

LN2 = 0.6931471805599453

NC = 2
NS = 16
NW = NC * NS
CHUNK = 80
EPW_A = E // NW
NCH_A = EPW_A // CHUNK
EPT_B = E // NS
NCH_B = EPT_B // CHUNK

ZROWS = 9216
TRASH = 9100
ZPT = ZROWS // NS
ZB = 288


def _ssp(x):
    return jnp.logaddexp(x, 0.0) - LN2


def _hx_body(nuc_ref, elec_ref, w_ref, b_ref, out_ref):
    out_ref[:, KER:] = jnp.zeros((N_NODES, KER), jnp.float32)
    out_ref[0:N_NUC, 0:KER] = nuc_ref[...]
    out_ref[N_NUC:, 0:KER] = (
        jnp.dot(elec_ref[...], w_ref[...], preferred_element_type=jnp.float32)
        + b_ref[...]
    )


def _hx(nuc, elec, hw, hb):
    return pl.pallas_call(
        _hx_body,
        out_shape=jax.ShapeDtypeStruct((N_NODES, 2 * KER), jnp.float32),
    )(nuc, elec, hw, hb.reshape(1, KER))


def _gather_body(hx_hbm, snd_hbm, gx_hbm, snd_v0, snd_v1, hxg_v0, hxg_v1,
                 sem0, sem1):
    cid = lax.axis_index("c")
    sid = lax.axis_index("s")
    wid = sid * NC + cid
    base = wid * EPW_A

    def _pair(k, _):
        off0 = base + (2 * k) * CHUNK
        off1 = off0 + CHUNK
        pltpu.sync_copy(snd_hbm.at[pl.ds(off0, CHUNK)], snd_v0)
        g0 = pltpu.async_copy(hx_hbm.at[snd_v0], hxg_v0, sem0)
        pltpu.sync_copy(snd_hbm.at[pl.ds(off1, CHUNK)], snd_v1)
        g1 = pltpu.async_copy(hx_hbm.at[snd_v1], hxg_v1, sem1)
        g0.wait()
        pltpu.sync_copy(hxg_v0, gx_hbm.at[pl.ds(off0, CHUNK), :])
        g1.wait()
        pltpu.sync_copy(hxg_v1, gx_hbm.at[pl.ds(off1, CHUNK), :])
        return 0

    lax.fori_loop(0, NCH_A // 2, _pair, 0)

    off = base + (NCH_A - 1) * CHUNK
    pltpu.sync_copy(snd_hbm.at[pl.ds(off, CHUNK)], snd_v0)
    pltpu.async_copy(hx_hbm.at[snd_v0], hxg_v0, sem0).wait()
    pltpu.sync_copy(hxg_v0, gx_hbm.at[pl.ds(off, CHUNK), :])


_sc_gather = functools.partial(
    pl.kernel,
    out_type=jax.ShapeDtypeStruct((E, 2 * KER), jnp.float32),
    mesh=plsc.VectorSubcoreMesh(core_axis_name="c", subcore_axis_name="s",
                                num_cores=NC, num_subcores=NS),
    scratch_types=[
        pltpu.VMEM((CHUNK,), jnp.int32),
        pltpu.VMEM((CHUNK,), jnp.int32),
        pltpu.VMEM((CHUNK, 2 * KER), jnp.float32),
        pltpu.VMEM((CHUNK, 2 * KER), jnp.float32),
        pltpu.SemaphoreType.DMA,
        pltpu.SemaphoreType.DMA,
    ],
)(_gather_body)


_BE = 2000


def _wemlp_body(dist_ref, etf_ref, w1_ref, b1_ref, w2_ref, b2s_ref, out_ref):
    x = dist_ref[...]
    h = jnp.dot(x, w1_ref[...], preferred_element_type=jnp.float32) + b1_ref[...]
    h = _ssp(h)
    et = etf_ref[...]
    col = lax.broadcasted_iota(jnp.int32, (1, 3 * DFD), 1)
    coltype = jnp.where(col < DFD, 3.0, jnp.where(col < 2 * DFD, 4.0, 1.0))
    h = jnp.where(et == coltype, h, 0.0)
    we = jnp.dot(h, w2_ref[...], preferred_element_type=jnp.float32)
    b2 = (
        jnp.where(et == 3.0, 1.0, 0.0) * b2s_ref[0:1, :]
        + jnp.where(et == 4.0, 1.0, 0.0) * b2s_ref[1:2, :]
        + jnp.where(et == 1.0, 1.0, 0.0) * b2s_ref[2:3, :]
    )
    out_ref[...] = we + b2


def _wemlp(dist, etf, w1c, b1c, w2c, b2s):
    nb = E // _BE
    return pl.pallas_call(
        _wemlp_body,
        grid=(nb,),
        in_specs=[
            pl.BlockSpec((_BE, DFD), lambda i: (i, 0)),
            pl.BlockSpec((_BE, 1), lambda i: (i, 0)),
            pl.BlockSpec((DFD, 3 * DFD), lambda i: (0, 0)),
            pl.BlockSpec((1, 3 * DFD), lambda i: (0, 0)),
            pl.BlockSpec((3 * DFD, KER), lambda i: (0, 0)),
            pl.BlockSpec((8, KER), lambda i: (0, 0)),
        ],
        out_specs=pl.BlockSpec((_BE, KER), lambda i: (i, 0)),
        out_shape=jax.ShapeDtypeStruct((E, KER), jnp.float32),
    )(dist, etf, w1c, b1c, w2c, b2s)


def _wproj_body(we_ref, etf_ref, gx_ref, g_ref, out_ref):
    et = etf_ref[...]
    weh = we_ref[...] * gx_ref[:, 0:KER]
    acc = jnp.zeros((_BE, EMB), jnp.float32)
    for t, tval in enumerate((3.0, 4.0, 1.0)):
        wt = jnp.where(et == tval, weh, 0.0)
        acc = acc + jnp.dot(wt, g_ref[t], preferred_element_type=jnp.float32)
    out_ref[...] = acc


def _wproj(we, etf, gx, gstk):
    nb = E // _BE
    return pl.pallas_call(
        _wproj_body,
        grid=(nb,),
        in_specs=[
            pl.BlockSpec((_BE, KER), lambda i: (i, 0)),
            pl.BlockSpec((_BE, 1), lambda i: (i, 0)),
            pl.BlockSpec((_BE, 2 * KER), lambda i: (i, 0)),
            pl.BlockSpec((3, KER, EMB), lambda i: (0, 0, 0)),
        ],
        out_specs=pl.BlockSpec((_BE, EMB), lambda i: (i, 0)),
        out_shape=jax.ShapeDtypeStruct((E, EMB), jnp.float32),
    )(we, etf, gx, gstk)


def _scatter_body(v_hbm, rcv_hbm, z_hbm, rcv_v, idx_v, vv_v, zeros_v, z_sh):
    cid = lax.axis_index("c")
    sid = lax.axis_index("s")
    wid = sid * NC + cid

    def _zero_row(r, _):
        for c in range(EMB // 16):
            zeros_v[r, pl.ds(c * 16, 16)] = jnp.zeros((16,), jnp.float32)
        return 0

    lax.fori_loop(0, ZB, _zero_row, 0)
    for k in range(ZPT // ZB):
        pltpu.sync_copy(zeros_v, z_sh.at[pl.ds(sid * ZPT + k * ZB, ZB)])
    plsc.subcore_barrier()

    base = wid * EPW_A

    def _chunk(cix, _):
        off = base + cix * CHUNK
        pltpu.sync_copy(rcv_hbm.at[pl.ds(off, CHUNK)], rcv_v)
        pltpu.sync_copy(v_hbm.at[pl.ds(off, CHUNK), :], vv_v)

        def _mkidx(i, _):
            sl = pl.ds(i * 16, 16)
            d = rcv_v[sl] - N_NUC
            idx_v[sl] = jnp.where((d >= 0) & (d < N_ELEC), d, TRASH)
            return 0

        lax.fori_loop(0, CHUNK // 16, _mkidx, 0)
        pltpu.sync_copy(vv_v, z_sh.at[idx_v], add=True)
        return 0

    lax.fori_loop(0, NCH_A, _chunk, 0)
    plsc.subcore_barrier()
    pltpu.sync_copy(z_sh.at[pl.ds(sid * ZPT, ZPT)],
                    z_hbm.at[cid, pl.ds(sid * ZPT, ZPT), :])


_sc_scatter = functools.partial(
    pl.kernel,
    out_type=jax.ShapeDtypeStruct((NC, ZROWS, EMB), jnp.float32),
    mesh=plsc.VectorSubcoreMesh(core_axis_name="c", subcore_axis_name="s",
                                num_cores=NC, num_subcores=NS),
    scratch_types=[
        pltpu.VMEM((CHUNK,), jnp.int32),
        pltpu.VMEM((CHUNK,), jnp.int32),
        pltpu.VMEM((CHUNK, EMB), jnp.float32),
        pltpu.VMEM((ZB, EMB), jnp.float32),
        pltpu.VMEM_SHARED((ZROWS, EMB), jnp.float32),
    ],
)(_scatter_body)


def _fin_body(elec_ref, z_ref, gb_ref, out_ref):
    out_ref[...] = (elec_ref[...] + z_ref[0, 0:N_ELEC, :]
                    + z_ref[1, 0:N_ELEC, :] + gb_ref[...])


def _fin(elec, z, gbias):
    return pl.pallas_call(
        _fin_body,
        out_shape=jax.ShapeDtypeStruct((N_ELEC, EMB), jnp.float32),
    )(elec, z, gbias)


def kernel(nuc, elec, dist, e_type, senders, receivers, params):
    w = params['w']
    g = params['g']
    w1c = jnp.concatenate([w['same']['W1'], w['anti']['W1'], w['n']['W1']], axis=1)
    b1c = jnp.concatenate([w['same']['b1'], w['anti']['b1'], w['n']['b1']]).reshape(1, 3 * DFD)
    w2c = jnp.concatenate([w['same']['W2'], w['anti']['W2'], w['n']['W2']], axis=0)
    b2s = jnp.concatenate([
        w['same']['b2'].reshape(1, KER),
        w['anti']['b2'].reshape(1, KER),
        w['n']['b2'].reshape(1, KER),
        jnp.zeros((5, KER), jnp.float32),
    ], axis=0)
    gstk = jnp.stack([g['same']['W'], g['anti']['W'], g['n']['W']])
    gbias = (g['same']['b'] + g['anti']['b'] + g['n']['b']).reshape(1, EMB)

    etf = e_type.astype(jnp.float32).reshape(E, 1)
    hx = _hx(nuc, elec, params['h']['W'], params['h']['b'])
    gx = _sc_gather(hx, senders.astype(jnp.int32))
    we = _wemlp(dist, etf, w1c, b1c, w2c, b2s)
    v = _wproj(we, etf, gx, gstk)
    z = _sc_scatter(v, receivers.astype(jnp.int32))
    return _fin(elec, z, gbias)


# speedup vs baseline: 1.0706x; 1.0706x over previous
import functools

import jax
import jax.numpy as jnp
from jax import lax
from jax.experimental import pallas as pl
from jax.experimental.pallas import tpu as pltpu
from jax.experimental.pallas import tpu_sc as plsc

N_NUC = 1000
N_ELEC = 9000
N_NODES = 10000
E = 320000
EMB = 128
KER = 64
DFD = 128

LN2 = 0.6931471805599453

NC = 2
NS = 16
NW = NC * NS
CHUNK = 80
EPW_A = E // NW
NCH_A = EPW_A // CHUNK
EPT_B = E // NS
NCH_B = EPT_B // CHUNK

ZROWS = 9216
TRASH = 9100
ZPT = ZROWS // NS
ZB = 288


def _ssp(x):
    return jnp.logaddexp(x, 0.0) - LN2


def _hx_body(nuc_ref, elec_ref, w_ref, b_ref, out_ref):
    out_ref[:, KER:] = jnp.zeros((N_NODES, KER), jnp.float32)
    out_ref[0:N_NUC, 0:KER] = nuc_ref[...]
    out_ref[N_NUC:, 0:KER] = (
        jnp.dot(elec_ref[...], w_ref[...], preferred_element_type=jnp.float32)
        + b_ref[...]
    )


def _hx(nuc, elec, hw, hb):
    return pl.pallas_call(
        _hx_body,
        out_shape=jax.ShapeDtypeStruct((N_NODES, 2 * KER), jnp.float32),
    )(nuc, elec, hw, hb.reshape(1, KER))


def _gather_body(hx_hbm, snd_hbm, gx_hbm, snd_v, hxg_v, sem):
    cid = lax.axis_index("c")
    sid = lax.axis_index("s")
    wid = sid * NC + cid
    base = wid * EPW_A

    def _chunk(cix, _):
        off = base + cix * CHUNK
        pltpu.sync_copy(snd_hbm.at[pl.ds(off, CHUNK)], snd_v)
        pltpu.async_copy(hx_hbm.at[snd_v], hxg_v, sem).wait()
        pltpu.sync_copy(hxg_v, gx_hbm.at[pl.ds(off, CHUNK), :])
        return 0

    lax.fori_loop(0, NCH_A, _chunk, 0)


_sc_gather = functools.partial(
    pl.kernel,
    out_type=jax.ShapeDtypeStruct((E, 2 * KER), jnp.float32),
    mesh=plsc.VectorSubcoreMesh(core_axis_name="c", subcore_axis_name="s",
                                num_cores=NC, num_subcores=NS),
    scratch_types=[
        pltpu.VMEM((CHUNK,), jnp.int32),
        pltpu.VMEM((CHUNK, 2 * KER), jnp.float32),
        pltpu.SemaphoreType.DMA,
    ],
)(_gather_body)


_BE = 4000


def _wemlp_body(dist_ref, etf_ref, w1_ref, b1_ref, w2_ref, b2s_ref, out_ref):
    x = dist_ref[...]
    h = jnp.dot(x, w1_ref[...], preferred_element_type=jnp.float32) + b1_ref[...]
    h = _ssp(h)
    et = etf_ref[...]
    col = lax.broadcasted_iota(jnp.int32, (1, 3 * DFD), 1)
    coltype = jnp.where(col < DFD, 3.0, jnp.where(col < 2 * DFD, 4.0, 1.0))
    h = jnp.where(et == coltype, h, 0.0)
    we = jnp.dot(h, w2_ref[...], preferred_element_type=jnp.float32)
    b2 = (
        jnp.where(et == 3.0, 1.0, 0.0) * b2s_ref[0:1, :]
        + jnp.where(et == 4.0, 1.0, 0.0) * b2s_ref[1:2, :]
        + jnp.where(et == 1.0, 1.0, 0.0) * b2s_ref[2:3, :]
    )
    out_ref[...] = we + b2


def _wemlp(dist, etf, w1c, b1c, w2c, b2s):
    nb = E // _BE
    return pl.pallas_call(
        _wemlp_body,
        grid=(nb,),
        in_specs=[
            pl.BlockSpec((_BE, DFD), lambda i: (i, 0)),
            pl.BlockSpec((_BE, 1), lambda i: (i, 0)),
            pl.BlockSpec((DFD, 3 * DFD), lambda i: (0, 0)),
            pl.BlockSpec((1, 3 * DFD), lambda i: (0, 0)),
            pl.BlockSpec((3 * DFD, KER), lambda i: (0, 0)),
            pl.BlockSpec((8, KER), lambda i: (0, 0)),
        ],
        out_specs=pl.BlockSpec((_BE, KER), lambda i: (i, 0)),
        out_shape=jax.ShapeDtypeStruct((E, KER), jnp.float32),
    )(dist, etf, w1c, b1c, w2c, b2s)


def _wproj_body(we_ref, etf_ref, gx_ref, g_ref, out_ref):
    et = etf_ref[...]
    weh = we_ref[...] * gx_ref[:, 0:KER]
    acc = jnp.zeros((_BE, EMB), jnp.float32)
    for t, tval in enumerate((3.0, 4.0, 1.0)):
        wt = jnp.where(et == tval, weh, 0.0)
        acc = acc + jnp.dot(wt, g_ref[t], preferred_element_type=jnp.float32)
    out_ref[...] = acc


def _wproj(we, etf, gx, gstk):
    nb = E // _BE
    return pl.pallas_call(
        _wproj_body,
        grid=(nb,),
        in_specs=[
            pl.BlockSpec((_BE, KER), lambda i: (i, 0)),
            pl.BlockSpec((_BE, 1), lambda i: (i, 0)),
            pl.BlockSpec((_BE, 2 * KER), lambda i: (i, 0)),
            pl.BlockSpec((3, KER, EMB), lambda i: (0, 0, 0)),
        ],
        out_specs=pl.BlockSpec((_BE, EMB), lambda i: (i, 0)),
        out_shape=jax.ShapeDtypeStruct((E, EMB), jnp.float32),
    )(we, etf, gx, gstk)


def _scatter_body(v_hbm, rcv_hbm, z_hbm, rcv_v, idx_v, vv_v, zeros_v, z_sh):
    cid = lax.axis_index("c")
    sid = lax.axis_index("s")
    wid = sid * NC + cid

    def _zero_row(r, _):
        for c in range(EMB // 16):
            zeros_v[r, pl.ds(c * 16, 16)] = jnp.zeros((16,), jnp.float32)
        return 0

    lax.fori_loop(0, ZB, _zero_row, 0)
    for k in range(ZPT // ZB):
        pltpu.sync_copy(zeros_v, z_sh.at[pl.ds(sid * ZPT + k * ZB, ZB)])
    plsc.subcore_barrier()

    base = wid * EPW_A

    def _chunk(cix, _):
        off = base + cix * CHUNK
        pltpu.sync_copy(rcv_hbm.at[pl.ds(off, CHUNK)], rcv_v)
        pltpu.sync_copy(v_hbm.at[pl.ds(off, CHUNK), :], vv_v)

        def _mkidx(i, _):
            sl = pl.ds(i * 16, 16)
            d = rcv_v[sl] - N_NUC
            idx_v[sl] = jnp.where((d >= 0) & (d < N_ELEC), d, TRASH)
            return 0

        lax.fori_loop(0, CHUNK // 16, _mkidx, 0)
        pltpu.sync_copy(vv_v, z_sh.at[idx_v], add=True)
        return 0

    lax.fori_loop(0, NCH_A, _chunk, 0)
    plsc.subcore_barrier()
    pltpu.sync_copy(z_sh.at[pl.ds(sid * ZPT, ZPT)],
                    z_hbm.at[cid, pl.ds(sid * ZPT, ZPT), :])


_sc_scatter = functools.partial(
    pl.kernel,
    out_type=jax.ShapeDtypeStruct((NC, ZROWS, EMB), jnp.float32),
    mesh=plsc.VectorSubcoreMesh(core_axis_name="c", subcore_axis_name="s",
                                num_cores=NC, num_subcores=NS),
    scratch_types=[
        pltpu.VMEM((CHUNK,), jnp.int32),
        pltpu.VMEM((CHUNK,), jnp.int32),
        pltpu.VMEM((CHUNK, EMB), jnp.float32),
        pltpu.VMEM((ZB, EMB), jnp.float32),
        pltpu.VMEM_SHARED((ZROWS, EMB), jnp.float32),
    ],
)(_scatter_body)


def _fin_body(elec_ref, z_ref, gb_ref, out_ref):
    out_ref[...] = (elec_ref[...] + z_ref[0, 0:N_ELEC, :]
                    + z_ref[1, 0:N_ELEC, :] + gb_ref[...])


def _fin(elec, z, gbias):
    return pl.pallas_call(
        _fin_body,
        out_shape=jax.ShapeDtypeStruct((N_ELEC, EMB), jnp.float32),
    )(elec, z, gbias)


def kernel(nuc, elec, dist, e_type, senders, receivers, params):
    w = params['w']
    g = params['g']
    w1c = jnp.concatenate([w['same']['W1'], w['anti']['W1'], w['n']['W1']], axis=1)
    b1c = jnp.concatenate([w['same']['b1'], w['anti']['b1'], w['n']['b1']]).reshape(1, 3 * DFD)
    w2c = jnp.concatenate([w['same']['W2'], w['anti']['W2'], w['n']['W2']], axis=0)
    b2s = jnp.concatenate([
        w['same']['b2'].reshape(1, KER),
        w['anti']['b2'].reshape(1, KER),
        w['n']['b2'].reshape(1, KER),
        jnp.zeros((5, KER), jnp.float32),
    ], axis=0)
    gstk = jnp.stack([g['same']['W'], g['anti']['W'], g['n']['W']])
    gbias = (g['same']['b'] + g['anti']['b'] + g['n']['b']).reshape(1, EMB)

    etf = e_type.astype(jnp.float32).reshape(E, 1)
    hx = _hx(nuc, elec, params['h']['W'], params['h']['b'])
    gx = _sc_gather(hx, senders.astype(jnp.int32))
    we = _wemlp(dist, etf, w1c, b1c, w2c, b2s)
    v = _wproj(we, etf, gx, gstk)
    z = _sc_scatter(v, receivers.astype(jnp.int32))
    return _fin(elec, z, gbias)


# speedup vs baseline: 1.0970x; 1.0247x over previous
import functools

import jax
import jax.numpy as jnp
from jax import lax
from jax.experimental import pallas as pl
from jax.experimental.pallas import tpu as pltpu
from jax.experimental.pallas import tpu_sc as plsc

N_NUC = 1000
N_ELEC = 9000
N_NODES = 10000
E = 320000
EMB = 128
KER = 64
DFD = 128

LN2 = 0.6931471805599453

NC = 2
NS = 16
NW = NC * NS
CHUNK = 80
EPW_A = E // NW
NCH_A = EPW_A // CHUNK
EPT_B = E // NS
NCH_B = EPT_B // CHUNK

ZROWS = 9216
TRASH = 9100
ZPT = ZROWS // NS
ZB = 288


def _ssp(x):
    return jnp.logaddexp(x, 0.0) - LN2


def _hx_body(nuc_ref, elec_ref, w_ref, b_ref, out_ref):
    out_ref[:, KER:] = jnp.zeros((N_NODES, KER), jnp.float32)
    out_ref[0:N_NUC, 0:KER] = nuc_ref[...]
    out_ref[N_NUC:, 0:KER] = (
        jnp.dot(elec_ref[...], w_ref[...], preferred_element_type=jnp.float32)
        + b_ref[...]
    )


def _hx(nuc, elec, hw, hb):
    return pl.pallas_call(
        _hx_body,
        out_shape=jax.ShapeDtypeStruct((N_NODES, 2 * KER), jnp.float32),
    )(nuc, elec, hw, hb.reshape(1, KER))


def _gather_body(hx_hbm, snd_hbm, gx_hbm, snd_v, hxg_v, sem):
    cid = lax.axis_index("c")
    sid = lax.axis_index("s")
    wid = sid * NC + cid
    base = wid * EPW_A

    def _chunk(cix, _):
        off = base + cix * CHUNK
        pltpu.sync_copy(snd_hbm.at[pl.ds(off, CHUNK)], snd_v)
        pltpu.async_copy(hx_hbm.at[snd_v], hxg_v, sem).wait()
        pltpu.sync_copy(hxg_v, gx_hbm.at[pl.ds(off, CHUNK), :])
        return 0

    lax.fori_loop(0, NCH_A, _chunk, 0)


_sc_gather = functools.partial(
    pl.kernel,
    out_type=jax.ShapeDtypeStruct((E, 2 * KER), jnp.float32),
    mesh=plsc.VectorSubcoreMesh(core_axis_name="c", subcore_axis_name="s",
                                num_cores=NC, num_subcores=NS),
    scratch_types=[
        pltpu.VMEM((CHUNK,), jnp.int32),
        pltpu.VMEM((CHUNK, 2 * KER), jnp.float32),
        pltpu.SemaphoreType.DMA,
    ],
)(_gather_body)


_BE = 8000


def _wemlp_body(dist_ref, etf_ref, w1_ref, b1_ref, w2_ref, b2s_ref, out_ref):
    x = dist_ref[...]
    h = jnp.dot(x, w1_ref[...], preferred_element_type=jnp.float32) + b1_ref[...]
    h = _ssp(h)
    et = etf_ref[...]
    col = lax.broadcasted_iota(jnp.int32, (1, 3 * DFD), 1)
    coltype = jnp.where(col < DFD, 3.0, jnp.where(col < 2 * DFD, 4.0, 1.0))
    h = jnp.where(et == coltype, h, 0.0)
    we = jnp.dot(h, w2_ref[...], preferred_element_type=jnp.float32)
    b2 = (
        jnp.where(et == 3.0, 1.0, 0.0) * b2s_ref[0:1, :]
        + jnp.where(et == 4.0, 1.0, 0.0) * b2s_ref[1:2, :]
        + jnp.where(et == 1.0, 1.0, 0.0) * b2s_ref[2:3, :]
    )
    out_ref[...] = we + b2


def _wemlp(dist, etf, w1c, b1c, w2c, b2s):
    nb = E // _BE
    return pl.pallas_call(
        _wemlp_body,
        grid=(nb,),
        in_specs=[
            pl.BlockSpec((_BE, DFD), lambda i: (i, 0)),
            pl.BlockSpec((_BE, 1), lambda i: (i, 0)),
            pl.BlockSpec((DFD, 3 * DFD), lambda i: (0, 0)),
            pl.BlockSpec((1, 3 * DFD), lambda i: (0, 0)),
            pl.BlockSpec((3 * DFD, KER), lambda i: (0, 0)),
            pl.BlockSpec((8, KER), lambda i: (0, 0)),
        ],
        out_specs=pl.BlockSpec((_BE, KER), lambda i: (i, 0)),
        out_shape=jax.ShapeDtypeStruct((E, KER), jnp.float32),
    )(dist, etf, w1c, b1c, w2c, b2s)


def _wproj_body(we_ref, etf_ref, gx_ref, g_ref, out_ref):
    et = etf_ref[...]
    weh = we_ref[...] * gx_ref[:, 0:KER]
    acc = jnp.zeros((_BE, EMB), jnp.float32)
    for t, tval in enumerate((3.0, 4.0, 1.0)):
        wt = jnp.where(et == tval, weh, 0.0)
        acc = acc + jnp.dot(wt, g_ref[t], preferred_element_type=jnp.float32)
    out_ref[...] = acc


def _wproj(we, etf, gx, gstk):
    nb = E // _BE
    return pl.pallas_call(
        _wproj_body,
        grid=(nb,),
        in_specs=[
            pl.BlockSpec((_BE, KER), lambda i: (i, 0)),
            pl.BlockSpec((_BE, 1), lambda i: (i, 0)),
            pl.BlockSpec((_BE, 2 * KER), lambda i: (i, 0)),
            pl.BlockSpec((3, KER, EMB), lambda i: (0, 0, 0)),
        ],
        out_specs=pl.BlockSpec((_BE, EMB), lambda i: (i, 0)),
        out_shape=jax.ShapeDtypeStruct((E, EMB), jnp.float32),
    )(we, etf, gx, gstk)


def _scatter_body(v_hbm, rcv_hbm, z_hbm, rcv_v, idx_v, vv_v, zeros_v, z_sh):
    cid = lax.axis_index("c")
    sid = lax.axis_index("s")
    wid = sid * NC + cid

    def _zero_row(r, _):
        for c in range(EMB // 16):
            zeros_v[r, pl.ds(c * 16, 16)] = jnp.zeros((16,), jnp.float32)
        return 0

    lax.fori_loop(0, ZB, _zero_row, 0)
    for k in range(ZPT // ZB):
        pltpu.sync_copy(zeros_v, z_sh.at[pl.ds(sid * ZPT + k * ZB, ZB)])
    plsc.subcore_barrier()

    base = wid * EPW_A

    def _chunk(cix, _):
        off = base + cix * CHUNK
        pltpu.sync_copy(rcv_hbm.at[pl.ds(off, CHUNK)], rcv_v)
        pltpu.sync_copy(v_hbm.at[pl.ds(off, CHUNK), :], vv_v)

        def _mkidx(i, _):
            sl = pl.ds(i * 16, 16)
            d = rcv_v[sl] - N_NUC
            idx_v[sl] = jnp.where((d >= 0) & (d < N_ELEC), d, TRASH)
            return 0

        lax.fori_loop(0, CHUNK // 16, _mkidx, 0)
        pltpu.sync_copy(vv_v, z_sh.at[idx_v], add=True)
        return 0

    lax.fori_loop(0, NCH_A, _chunk, 0)
    plsc.subcore_barrier()
    pltpu.sync_copy(z_sh.at[pl.ds(sid * ZPT, ZPT)],
                    z_hbm.at[cid, pl.ds(sid * ZPT, ZPT), :])


_sc_scatter = functools.partial(
    pl.kernel,
    out_type=jax.ShapeDtypeStruct((NC, ZROWS, EMB), jnp.float32),
    mesh=plsc.VectorSubcoreMesh(core_axis_name="c", subcore_axis_name="s",
                                num_cores=NC, num_subcores=NS),
    scratch_types=[
        pltpu.VMEM((CHUNK,), jnp.int32),
        pltpu.VMEM((CHUNK,), jnp.int32),
        pltpu.VMEM((CHUNK, EMB), jnp.float32),
        pltpu.VMEM((ZB, EMB), jnp.float32),
        pltpu.VMEM_SHARED((ZROWS, EMB), jnp.float32),
    ],
)(_scatter_body)


def _fin_body(elec_ref, z_ref, gb_ref, out_ref):
    out_ref[...] = (elec_ref[...] + z_ref[0, 0:N_ELEC, :]
                    + z_ref[1, 0:N_ELEC, :] + gb_ref[...])


def _fin(elec, z, gbias):
    return pl.pallas_call(
        _fin_body,
        out_shape=jax.ShapeDtypeStruct((N_ELEC, EMB), jnp.float32),
    )(elec, z, gbias)


def kernel(nuc, elec, dist, e_type, senders, receivers, params):
    w = params['w']
    g = params['g']
    w1c = jnp.concatenate([w['same']['W1'], w['anti']['W1'], w['n']['W1']], axis=1)
    b1c = jnp.concatenate([w['same']['b1'], w['anti']['b1'], w['n']['b1']]).reshape(1, 3 * DFD)
    w2c = jnp.concatenate([w['same']['W2'], w['anti']['W2'], w['n']['W2']], axis=0)
    b2s = jnp.concatenate([
        w['same']['b2'].reshape(1, KER),
        w['anti']['b2'].reshape(1, KER),
        w['n']['b2'].reshape(1, KER),
        jnp.zeros((5, KER), jnp.float32),
    ], axis=0)
    gstk = jnp.stack([g['same']['W'], g['anti']['W'], g['n']['W']])
    gbias = (g['same']['b'] + g['anti']['b'] + g['n']['b']).reshape(1, EMB)

    etf = e_type.astype(jnp.float32).reshape(E, 1)
    hx = _hx(nuc, elec, params['h']['W'], params['h']['b'])
    gx = _sc_gather(hx, senders.astype(jnp.int32))
    we = _wemlp(dist, etf, w1c, b1c, w2c, b2s)
    v = _wproj(we, etf, gx, gstk)
    z = _sc_scatter(v, receivers.astype(jnp.int32))
    return _fin(elec, z, gbias)
